# baseline (speedup 1.0000x reference)
import functools

import jax
import jax.numpy as jnp
from jax import lax
from jax.experimental import pallas as pl
from jax.experimental.pallas import tpu as pltpu
from jax.experimental.pallas import tpu_sc as plsc

F32 = jnp.float32
_BN_SCALE = 1.0 / (1.0 + 1e-5) ** 0.5


def _node_body(batch_ref, h_ref, vn_ref, hin_ref, gsum_ref):
    i = pl.program_id(0)
    b = batch_ref[...]
    iot = lax.broadcasted_iota(jnp.int32, (b.shape[0], vn_ref.shape[0]), 1)
    oh = (b == iot).astype(F32)
    hin = h_ref[...] + jnp.dot(oh, vn_ref[...], preferred_element_type=F32)
    hin_ref[...] = hin
    part = lax.dot_general(oh, hin, (((0,), (0,)), ((), ())),
                           preferred_element_type=F32)

    @pl.when(i == 0)
    def _():
        gsum_ref[...] = part

    @pl.when(i != 0)
    def _():
        gsum_ref[...] = gsum_ref[...] + part


def _node_call(batch2d, h, vn, bn=200):
    n, d = h.shape
    ng = vn.shape[0]
    grid = n // bn
    return pl.pallas_call(
        _node_body,
        grid=(grid,),
        in_specs=[
            pl.BlockSpec((bn, 1), lambda i: (i, 0)),
            pl.BlockSpec((bn, d), lambda i: (i, 0)),
            pl.BlockSpec((ng, d), lambda i: (0, 0)),
        ],
        out_specs=[
            pl.BlockSpec((bn, d), lambda i: (i, 0)),
            pl.BlockSpec((ng, d), lambda i: (0, 0)),
        ],
        out_shape=[
            jax.ShapeDtypeStruct((n, d), F32),
            jax.ShapeDtypeStruct((ng, d), F32),
        ],
    )(batch2d, h, vn)


def _edge_body(ea_ref, we_ref, be_ref, out_ref):
    out_ref[...] = (
        jnp.dot(ea_ref[...], we_ref[...], preferred_element_type=F32)
        + be_ref[...]
    ).astype(jnp.bfloat16)


def _edge_call(edge_attr, we, be2d, be_blk=2000):
    e, ed = edge_attr.shape
    d = we.shape[1]
    grid = e // be_blk
    return pl.pallas_call(
        _edge_body,
        grid=(grid,),
        in_specs=[
            pl.BlockSpec((be_blk, ed), lambda i: (i, 0)),
            pl.BlockSpec((ed, d), lambda i: (0, 0)),
            pl.BlockSpec((1, d), lambda i: (0, 0)),
        ],
        out_specs=pl.BlockSpec((be_blk, d), lambda i: (i, 0)),
        out_shape=jax.ShapeDtypeStruct((e, d), jnp.bfloat16),
    )(edge_attr, we, be2d)


def _mlp_body(relu_out, hin_ref, a0_ref, a1_ref, eps_ref, w1_ref, b1_ref,
              g1_ref, beta1_ref, w2_ref, b2_ref, g_ref, bb_ref, out_ref):
    hin = hin_ref[...]
    pre = (1.0 + eps_ref[0, 0]) * hin + a0_ref[...] + a1_ref[...]
    t = jnp.dot(pre, w1_ref[...], preferred_element_type=F32) + b1_ref[...]
    t = jnp.maximum(t * _BN_SCALE * g1_ref[...] + beta1_ref[...], 0.0)
    h2 = jnp.dot(t, w2_ref[...], preferred_element_type=F32) + b2_ref[...]
    h2 = h2 * _BN_SCALE * g_ref[...] + bb_ref[...]
    if relu_out:
        h2 = jnp.maximum(h2, 0.0)
    out_ref[...] = h2 + hin


def _mlp_call(hin, aggout, eps11, w1, b1, g1, beta1, w2, b2, g, bb,
              relu_out, bn=200):
    n, d = hin.shape
    d2 = w1.shape[1]
    grid = n // bn
    nblk = n // bn
    return pl.pallas_call(
        functools.partial(_mlp_body, relu_out),
        grid=(grid,),
        in_specs=[
            pl.BlockSpec((bn, d), lambda i: (i, 0)),
            pl.BlockSpec((bn, d), lambda i: (i, 0)),
            pl.BlockSpec((bn, d), lambda i, nb=nblk: (i + nb, 0)),
            pl.BlockSpec((1, 1), lambda i: (0, 0)),
            pl.BlockSpec((d, d2), lambda i: (0, 0)),
            pl.BlockSpec((1, d2), lambda i: (0, 0)),
            pl.BlockSpec((1, d2), lambda i: (0, 0)),
            pl.BlockSpec((1, d2), lambda i: (0, 0)),
            pl.BlockSpec((d2, d), lambda i: (0, 0)),
            pl.BlockSpec((1, d), lambda i: (0, 0)),
            pl.BlockSpec((1, d), lambda i: (0, 0)),
            pl.BlockSpec((1, d), lambda i: (0, 0)),
        ],
        out_specs=pl.BlockSpec((bn, d), lambda i: (i, 0)),
        out_shape=jax.ShapeDtypeStruct((n, d), F32),
    )(hin, aggout, aggout, eps11, w1, b1, g1, beta1, w2, b2, g, bb)


def _vn_body(gsum_ref, vn_ref, w1_ref, b1_ref, g1_ref, beta1_ref, w2_ref,
             b2_ref, g2_ref, beta2_ref, out_ref):
    vt = gsum_ref[...] + vn_ref[...]
    u = jnp.dot(vt, w1_ref[...], preferred_element_type=F32) + b1_ref[...]
    u = jnp.maximum(u * _BN_SCALE * g1_ref[...] + beta1_ref[...], 0.0)
    u = jnp.dot(u, w2_ref[...], preferred_element_type=F32) + b2_ref[...]
    u = jnp.maximum(u * _BN_SCALE * g2_ref[...] + beta2_ref[...], 0.0)
    out_ref[...] = vn_ref[...] + u


def _vn_call(gsum, vn, w1, b1, g1, beta1, w2, b2, g2, beta2):
    ng, d = vn.shape
    d2 = w1.shape[1]
    return pl.pallas_call(
        _vn_body,
        out_shape=jax.ShapeDtypeStruct((ng, d), F32),
    )(gsum, vn, w1, b1.reshape(1, d2), g1.reshape(1, d2),
      beta1.reshape(1, d2), w2, b2.reshape(1, d), g2.reshape(1, d),
      beta2.reshape(1, d))


_CS = 64
_GRP = 32


def _make_sc_msg(n, d, e):
    cs = _CS
    grp = _GRP
    nchunk = e // cs
    nq = d // 16
    mesh = plsc.VectorSubcoreMesh(core_axis_name="c", subcore_axis_name="s")
    per_w = -(-nchunk // 32)
    per_w = ((per_w + grp - 1) // grp) * grp
    tail_w = nchunk - 31 * per_w
    assert 0 < tail_w <= per_w
    nfull = n // cs
    ztail = n - nfull * cs
    zfull_k = nfull // 16
    zrem = nfull % 16

    @functools.partial(
        pl.kernel,
        out_type=jax.ShapeDtypeStruct((2 * n, d), F32),
        mesh=mesh,
        compiler_params=pltpu.CompilerParams(needs_layout_passes=False),
        scratch_types=[
            pltpu.VMEM((grp, cs), jnp.int32),
            pltpu.VMEM((grp, cs), jnp.int32),
            pltpu.VMEM((cs, d), F32),
            pltpu.VMEM((cs, d), F32),
            pltpu.VMEM((cs, d), F32),
            pltpu.VMEM((cs // 2, d), jnp.int32),
            pltpu.VMEM((cs // 2, d), jnp.int32),
            pltpu.VMEM_SHARED((n, d), F32),
            pltpu.SemaphoreType.DMA,
            pltpu.SemaphoreType.DMA,
            pltpu.SemaphoreType.DMA,
            pltpu.SemaphoreType.DMA,
            pltpu.SemaphoreType.DMA,
            pltpu.SemaphoreType.DMA,
            pltpu.SemaphoreType.DMA,
            pltpu.SemaphoreType.DMA,
        ],
    )
    def sc_msg(hin_hbm, emb_hbm, src_hbm, dst_hbm, out_hbm,
               srcblk, dstblk, r0, r1, r2, e0, e1, agg,
               gs0, gs1, gs2, es0, es1, ss0, ss1, ss2):
        c = lax.axis_index("c")
        s = lax.axis_index("s")
        w = c * 16 + s
        rbufs = (r0, r1, r2)
        ebufs = (e0, e1)
        gsems = (gs0, gs1, gs2)
        esems = (es0, es1)
        ssems = (ss0, ss1, ss2)

        zv = jnp.zeros((16,), F32)

        def zbody(r, carry):
            for q in range(nq):
                r0[r, pl.ds(q * 16, 16)] = zv
            return carry

        lax.fori_loop(0, cs, zbody, 0)

        nz = zfull_k + jnp.where(s < zrem, 1, 0)

        def zcopy(k, carry):
            z = s + 16 * k
            pltpu.sync_copy(r0.at[pl.ds(0, cs)], agg.at[pl.ds(z * cs, cs)])
            return carry

        lax.fori_loop(0, nz, zcopy, 0)

        @pl.when(s == 0)
        def _():
            pltpu.sync_copy(r0.at[pl.ds(0, ztail)],
                            agg.at[pl.ds(nfull * cs, ztail)])

        plsc.subcore_barrier()

        base = w * per_w
        nk = jnp.where(w == 31, tail_w, per_w)

        def issue_gather(m, j):
            return pltpu.async_copy(hin_hbm.at[srcblk.at[m]],
                                    rbufs[m % 3], gsems[m % 3])

        def issue_emb(m, j):
            return pltpu.async_copy(emb_hbm.at[pl.ds(j * (cs // 2), cs // 2)],
                                    ebufs[m % 2], esems[m % 2])

        def gbody(g, carry):
            g0 = base + g * grp
            pltpu.sync_copy(src_hbm.at[pl.ds(g0, grp)], srcblk)
            pltpu.sync_copy(dst_hbm.at[pl.ds(g0, grp)], dstblk)
            nin = jnp.minimum(grp, nk - g * grp)

            issue_gather(0, g0)
            issue_emb(0, g0)

            @pl.when(1 < nin)
            def _():
                issue_gather(1, g0 + 1)
                issue_emb(1, g0 + 1)

            for kk in range(grp):
                rb = rbufs[kk % 3]
                eb = ebufs[kk % 2]

                @pl.when(kk < nin)
                def _(kk=kk, rb=rb, eb=eb):
                    pltpu.make_async_copy(
                        hin_hbm.at[srcblk.at[kk]], rb, gsems[kk % 3]).wait()
                    pltpu.make_async_copy(
                        emb_hbm.at[pl.ds((base + g * grp + kk) * (cs // 2),
                                         cs // 2)],
                        eb, esems[kk % 2]).wait()
                    if kk >= 1:
                        pltpu.make_async_copy(
                            rbufs[(kk - 1) % 3],
                            agg.at[dstblk.at[kk - 1]],
                            ssems[(kk - 1) % 3]).wait()

                @pl.when(jnp.logical_and(kk < nin, kk + 2 < nin))
                def _(kk=kk):
                    issue_gather(kk + 2, g0 + kk + 2)

                @pl.when(kk < nin)
                def _(kk=kk, rb=rb, eb=eb):
                    hi_mask = jnp.int32(-65536)

                    def cbody(r2, cc):
                        for rr in range(2):
                            row = 2 * r2 + rr
                            for q in range(d // 32):
                                y = eb[r2, pl.ds(rr * (d // 2) + q * 16, 16)]
                                ea2 = plsc.bitcast(y << 16, F32)
                                eb2 = plsc.bitcast(y & hi_mask, F32)
                                sla = pl.ds(q * 32, 16)
                                slb = pl.ds(q * 32 + 16, 16)
                                rb[row, sla] = jnp.maximum(
                                    rb[row, sla] + ea2, 0.0)
                                rb[row, slb] = jnp.maximum(
                                    rb[row, slb] + eb2, 0.0)
                        return cc

                    lax.fori_loop(0, cs // 2, cbody, 0)
                    if kk + 2 < grp:
                        @pl.when(kk + 2 < nin)
                        def _():
                            issue_emb(kk + 2, g0 + kk + 2)
                    pltpu.async_copy(rb, agg.at[dstblk.at[kk]],
                                     ssems[kk % 3], add=True)

                @pl.when(kk + 1 == nin)
                def _(kk=kk):
                    pltpu.make_async_copy(
                        rbufs[kk % 3], agg.at[dstblk.at[kk]],
                        ssems[kk % 3]).wait()

            return carry

        lax.fori_loop(0, (nk + grp - 1) // grp, gbody, 0)
        plsc.subcore_barrier()

        def wcopy(k, carry):
            z = s + 16 * k
            pltpu.sync_copy(agg.at[pl.ds(z * cs, cs)],
                            out_hbm.at[pl.ds(c * n + z * cs, cs)])
            return carry

        lax.fori_loop(0, nz, wcopy, 0)

        @pl.when(s == 0)
        def _():
            pltpu.sync_copy(agg.at[pl.ds(nfull * cs, ztail)],
                            out_hbm.at[pl.ds(c * n + nfull * cs, ztail)])

    return sc_msg


def kernel(input_feature, edge_index, edge_attr, batch, vn_emb, conv_We,
           conv_be, conv_eps, conv_W1, conv_b1, conv_g1, conv_beta1, conv_W2,
           conv_b2, bn_g, bn_b, vW1, vb1, vg1, vbeta1, vW2, vb2, vg2,
           vbeta2):
    n, d = input_feature.shape
    e = edge_index.shape[1]
    ng = 256
    num_layers = conv_We.shape[0]

    nchunk = e // _CS
    per_w = ((-(-nchunk // 32) + _GRP - 1) // _GRP) * _GRP
    pad_rows = 32 * per_w - nchunk
    src2d = jnp.pad(edge_index[0].reshape(nchunk, _CS), ((0, pad_rows), (0, 0)))
    dst2d = jnp.pad(edge_index[1].reshape(nchunk, _CS), ((0, pad_rows), (0, 0)))
    batch2d = batch.reshape(n, 1)
    vn = jnp.tile(vn_emb[0][None, :], (ng, 1))

    rho_list = []
    for g in range(d // 32):
        for i in range(16):
            rho_list.extend((32 * g + i, 32 * g + 16 + i))

    sc_msg = _make_sc_msg(n, d, e)

    def _pack_i32(x):
        m, dd = x.shape
        xi = lax.bitcast_convert_type(x.reshape(m, dd // 2, 2), jnp.int32)
        return xi.reshape(m // 2, dd)

    rho_np = jnp.asarray(rho_list)
    embs = [_pack_i32(_edge_call(edge_attr, conv_We[l][:, rho_np],
                                 conv_be[l][rho_np].reshape(1, d)))
            for l in range(num_layers)]

    h = input_feature
    for l in range(num_layers):
        hin, gsum = _node_call(batch2d, h, vn)
        aggout = sc_msg(hin, embs[l], src2d, dst2d)
        h = _mlp_call(
            hin, aggout, conv_eps[l].reshape(1, 1), conv_W1[l], conv_b1[l].reshape(1, -1),
            conv_g1[l].reshape(1, -1), conv_beta1[l].reshape(1, -1), conv_W2[l],
            conv_b2[l].reshape(1, d), bn_g[l].reshape(1, d), bn_b[l].reshape(1, d),
            relu_out=(l < num_layers - 1),
        )
        if l < num_layers - 1:
            vn = _vn_call(gsum, vn, vW1[l], vb1[l], vg1[l], vbeta1[l],
                          vW2[l], vb2[l], vg2[l], vbeta2[l])
    return h


# speedup vs baseline: 3.5316x; 3.5316x over previous
import functools

import jax
import jax.numpy as jnp
from jax import lax
from jax.experimental import pallas as pl
from jax.experimental.pallas import tpu as pltpu
from jax.experimental.pallas import tpu_sc as plsc

F32 = jnp.float32
_BN_SCALE = 1.0 / (1.0 + 1e-5) ** 0.5


def _node_body(batch_ref, h_ref, vn_ref, hin_ref, gsum_ref):
    i = pl.program_id(0)
    b = batch_ref[...]
    iot = lax.broadcasted_iota(jnp.int32, (b.shape[0], vn_ref.shape[0]), 1)
    oh = (b == iot).astype(F32)
    hin = h_ref[...] + jnp.dot(oh, vn_ref[...], preferred_element_type=F32)
    hin_ref[...] = hin
    part = lax.dot_general(oh, hin, (((0,), (0,)), ((), ())),
                           preferred_element_type=F32)

    @pl.when(i == 0)
    def _():
        gsum_ref[...] = part

    @pl.when(i != 0)
    def _():
        gsum_ref[...] = gsum_ref[...] + part


def _node_call(batch2d, h, vn, bn=200):
    n, d = h.shape
    ng = vn.shape[0]
    grid = n // bn
    return pl.pallas_call(
        _node_body,
        grid=(grid,),
        in_specs=[
            pl.BlockSpec((bn, 1), lambda i: (i, 0)),
            pl.BlockSpec((bn, d), lambda i: (i, 0)),
            pl.BlockSpec((ng, d), lambda i: (0, 0)),
        ],
        out_specs=[
            pl.BlockSpec((bn, d), lambda i: (i, 0)),
            pl.BlockSpec((ng, d), lambda i: (0, 0)),
        ],
        out_shape=[
            jax.ShapeDtypeStruct((n, d), F32),
            jax.ShapeDtypeStruct((ng, d), F32),
        ],
    )(batch2d, h, vn)


def _edge_body(ea_ref, we_ref, be_ref, out_ref):
    out_ref[...] = (
        jnp.dot(ea_ref[...], we_ref[...], preferred_element_type=F32)
        + be_ref[...]
    )


def _edge_call(edge_attr, we, be2d, be_blk=2000):
    e, ed = edge_attr.shape
    d = we.shape[1]
    grid = e // be_blk
    return pl.pallas_call(
        _edge_body,
        grid=(grid,),
        in_specs=[
            pl.BlockSpec((be_blk, ed), lambda i: (i, 0)),
            pl.BlockSpec((ed, d), lambda i: (0, 0)),
            pl.BlockSpec((1, d), lambda i: (0, 0)),
        ],
        out_specs=pl.BlockSpec((be_blk, d), lambda i: (i, 0)),
        out_shape=jax.ShapeDtypeStruct((e, d), F32),
    )(edge_attr, we, be2d)


def _mlp_body(relu_out, hin_ref, a0_ref, a1_ref, eps_ref, w1_ref, b1_ref,
              g1_ref, beta1_ref, w2_ref, b2_ref, g_ref, bb_ref, out_ref):
    hin = hin_ref[...]
    pre = (1.0 + eps_ref[0, 0]) * hin + a0_ref[...] + a1_ref[...]
    t = jnp.dot(pre, w1_ref[...], preferred_element_type=F32) + b1_ref[...]
    t = jnp.maximum(t * _BN_SCALE * g1_ref[...] + beta1_ref[...], 0.0)
    h2 = jnp.dot(t, w2_ref[...], preferred_element_type=F32) + b2_ref[...]
    h2 = h2 * _BN_SCALE * g_ref[...] + bb_ref[...]
    if relu_out:
        h2 = jnp.maximum(h2, 0.0)
    out_ref[...] = h2 + hin


def _mlp_call(hin, aggout, eps11, w1, b1, g1, beta1, w2, b2, g, bb,
              relu_out, bn=200):
    n, d = hin.shape
    d2 = w1.shape[1]
    grid = n // bn
    nblk = n // bn
    return pl.pallas_call(
        functools.partial(_mlp_body, relu_out),
        grid=(grid,),
        in_specs=[
            pl.BlockSpec((bn, d), lambda i: (i, 0)),
            pl.BlockSpec((bn, d), lambda i: (i, 0)),
            pl.BlockSpec((bn, d), lambda i, nb=nblk: (i + nb, 0)),
            pl.BlockSpec((1, 1), lambda i: (0, 0)),
            pl.BlockSpec((d, d2), lambda i: (0, 0)),
            pl.BlockSpec((1, d2), lambda i: (0, 0)),
            pl.BlockSpec((1, d2), lambda i: (0, 0)),
            pl.BlockSpec((1, d2), lambda i: (0, 0)),
            pl.BlockSpec((d2, d), lambda i: (0, 0)),
            pl.BlockSpec((1, d), lambda i: (0, 0)),
            pl.BlockSpec((1, d), lambda i: (0, 0)),
            pl.BlockSpec((1, d), lambda i: (0, 0)),
        ],
        out_specs=pl.BlockSpec((bn, d), lambda i: (i, 0)),
        out_shape=jax.ShapeDtypeStruct((n, d), F32),
    )(hin, aggout, aggout, eps11, w1, b1, g1, beta1, w2, b2, g, bb)


def _vn_body(gsum_ref, vn_ref, w1_ref, b1_ref, g1_ref, beta1_ref, w2_ref,
             b2_ref, g2_ref, beta2_ref, out_ref):
    vt = gsum_ref[...] + vn_ref[...]
    u = jnp.dot(vt, w1_ref[...], preferred_element_type=F32) + b1_ref[...]
    u = jnp.maximum(u * _BN_SCALE * g1_ref[...] + beta1_ref[...], 0.0)
    u = jnp.dot(u, w2_ref[...], preferred_element_type=F32) + b2_ref[...]
    u = jnp.maximum(u * _BN_SCALE * g2_ref[...] + beta2_ref[...], 0.0)
    out_ref[...] = vn_ref[...] + u


def _vn_call(gsum, vn, w1, b1, g1, beta1, w2, b2, g2, beta2):
    ng, d = vn.shape
    d2 = w1.shape[1]
    return pl.pallas_call(
        _vn_body,
        out_shape=jax.ShapeDtypeStruct((ng, d), F32),
    )(gsum, vn, w1, b1.reshape(1, d2), g1.reshape(1, d2),
      beta1.reshape(1, d2), w2, b2.reshape(1, d), g2.reshape(1, d),
      beta2.reshape(1, d))


_CS = 64
_GRP = 32


def _make_sc_msg(n, d, e):
    cs = _CS
    grp = _GRP
    nchunk = e // cs
    nq = d // 16
    mesh = plsc.VectorSubcoreMesh(core_axis_name="c", subcore_axis_name="s")
    per_w = -(-nchunk // 32)
    per_w = ((per_w + grp - 1) // grp) * grp
    tail_w = nchunk - 31 * per_w
    assert 0 < tail_w <= per_w
    nfull = n // cs
    ztail = n - nfull * cs
    zfull_k = nfull // 16
    zrem = nfull % 16

    @functools.partial(
        pl.kernel,
        out_type=jax.ShapeDtypeStruct((2 * n, d), F32),
        mesh=mesh,
        scratch_types=[
            pltpu.VMEM((grp, cs), jnp.int32),
            pltpu.VMEM((grp, cs), jnp.int32),
            pltpu.VMEM((cs, d), F32),
            pltpu.VMEM((cs, d), F32),
            pltpu.VMEM((cs, d), F32),
            pltpu.VMEM((cs, d), F32),
            pltpu.VMEM((cs, d), F32),
            pltpu.VMEM_SHARED((n, d), F32),
            pltpu.SemaphoreType.DMA,
            pltpu.SemaphoreType.DMA,
            pltpu.SemaphoreType.DMA,
            pltpu.SemaphoreType.DMA,
            pltpu.SemaphoreType.DMA,
            pltpu.SemaphoreType.DMA,
            pltpu.SemaphoreType.DMA,
            pltpu.SemaphoreType.DMA,
        ],
    )
    def sc_msg(hin_hbm, emb_hbm, src_hbm, dst_hbm, out_hbm,
               srcblk, dstblk, r0, r1, r2, e0, e1, agg,
               gs0, gs1, gs2, es0, es1, ss0, ss1, ss2):
        c = lax.axis_index("c")
        s = lax.axis_index("s")
        w = c * 16 + s
        rbufs = (r0, r1, r2)
        ebufs = (e0, e1)
        gsems = (gs0, gs1, gs2)
        esems = (es0, es1)
        ssems = (ss0, ss1, ss2)

        zv = jnp.zeros((16,), F32)

        def zbody(r, carry):
            for q in range(nq):
                r0[r, pl.ds(q * 16, 16)] = zv
            return carry

        lax.fori_loop(0, cs, zbody, 0)

        nz = zfull_k + jnp.where(s < zrem, 1, 0)

        def zcopy(k, carry):
            z = s + 16 * k
            pltpu.sync_copy(r0.at[pl.ds(0, cs)], agg.at[pl.ds(z * cs, cs)])
            return carry

        lax.fori_loop(0, nz, zcopy, 0)

        @pl.when(s == 0)
        def _():
            pltpu.sync_copy(r0.at[pl.ds(0, ztail)],
                            agg.at[pl.ds(nfull * cs, ztail)])

        plsc.subcore_barrier()

        base = w * per_w
        nk = jnp.where(w == 31, tail_w, per_w)

        def issue_gather(m, j):
            return pltpu.async_copy(hin_hbm.at[srcblk.at[m]],
                                    rbufs[m % 3], gsems[m % 3])

        def issue_emb(m, j):
            return pltpu.async_copy(emb_hbm.at[pl.ds(j * cs, cs)],
                                    ebufs[m % 2], esems[m % 2])

        def gbody(g, carry):
            g0 = base + g * grp
            pltpu.sync_copy(src_hbm.at[pl.ds(g0, grp)], srcblk)
            pltpu.sync_copy(dst_hbm.at[pl.ds(g0, grp)], dstblk)
            nin = jnp.minimum(grp, nk - g * grp)

            issue_gather(0, g0)
            issue_emb(0, g0)

            @pl.when(1 < nin)
            def _():
                issue_gather(1, g0 + 1)
                issue_emb(1, g0 + 1)

            for kk in range(grp):
                rb = rbufs[kk % 3]
                eb = ebufs[kk % 2]

                @pl.when(kk < nin)
                def _(kk=kk, rb=rb, eb=eb):
                    pltpu.make_async_copy(
                        hin_hbm.at[srcblk.at[kk]], rb, gsems[kk % 3]).wait()
                    pltpu.make_async_copy(
                        emb_hbm.at[pl.ds((base + g * grp + kk) * cs, cs)],
                        eb, esems[kk % 2]).wait()
                    if kk >= 1:
                        pltpu.make_async_copy(
                            rbufs[(kk - 1) % 3],
                            agg.at[dstblk.at[kk - 1]],
                            ssems[(kk - 1) % 3]).wait()

                @pl.when(jnp.logical_and(kk < nin, kk + 2 < nin))
                def _(kk=kk):
                    issue_gather(kk + 2, g0 + kk + 2)

                @pl.when(kk < nin)
                def _(kk=kk, rb=rb, eb=eb):
                    def cbody(r2, cc):
                        for rr in range(2):
                            row = 2 * r2 + rr
                            for q in range(nq):
                                sl = pl.ds(q * 16, 16)
                                rb[row, sl] = jnp.maximum(
                                    rb[row, sl] + eb[row, sl], 0.0)
                        return cc

                    lax.fori_loop(0, cs // 2, cbody, 0)
                    if kk + 2 < grp:
                        @pl.when(kk + 2 < nin)
                        def _():
                            issue_emb(kk + 2, g0 + kk + 2)
                    pltpu.async_copy(rb, agg.at[dstblk.at[kk]],
                                     ssems[kk % 3], add=True)

                @pl.when(kk + 1 == nin)
                def _(kk=kk):
                    pltpu.make_async_copy(
                        rbufs[kk % 3], agg.at[dstblk.at[kk]],
                        ssems[kk % 3]).wait()

            return carry

        lax.fori_loop(0, (nk + grp - 1) // grp, gbody, 0)
        plsc.subcore_barrier()

        def wcopy(k, carry):
            z = s + 16 * k
            pltpu.sync_copy(agg.at[pl.ds(z * cs, cs)],
                            out_hbm.at[pl.ds(c * n + z * cs, cs)])
            return carry

        lax.fori_loop(0, nz, wcopy, 0)

        @pl.when(s == 0)
        def _():
            pltpu.sync_copy(agg.at[pl.ds(nfull * cs, ztail)],
                            out_hbm.at[pl.ds(c * n + nfull * cs, ztail)])

    return sc_msg


def kernel(input_feature, edge_index, edge_attr, batch, vn_emb, conv_We,
           conv_be, conv_eps, conv_W1, conv_b1, conv_g1, conv_beta1, conv_W2,
           conv_b2, bn_g, bn_b, vW1, vb1, vg1, vbeta1, vW2, vb2, vg2,
           vbeta2):
    n, d = input_feature.shape
    e = edge_index.shape[1]
    ng = 256
    num_layers = conv_We.shape[0]

    nchunk = e // _CS
    per_w = ((-(-nchunk // 32) + _GRP - 1) // _GRP) * _GRP
    pad_rows = 32 * per_w - nchunk
    src2d = jnp.pad(edge_index[0].reshape(nchunk, _CS), ((0, pad_rows), (0, 0)))
    dst2d = jnp.pad(edge_index[1].reshape(nchunk, _CS), ((0, pad_rows), (0, 0)))
    batch2d = batch.reshape(n, 1)
    vn = jnp.tile(vn_emb[0][None, :], (ng, 1))

    sc_msg = _make_sc_msg(n, d, e)

    embs = [_edge_call(edge_attr, conv_We[l], conv_be[l].reshape(1, d))
            for l in range(num_layers)]

    h = input_feature
    for l in range(num_layers):
        hin, gsum = _node_call(batch2d, h, vn)
        aggout = sc_msg(hin, embs[l], src2d, dst2d)
        h = _mlp_call(
            hin, aggout, conv_eps[l].reshape(1, 1), conv_W1[l], conv_b1[l].reshape(1, -1),
            conv_g1[l].reshape(1, -1), conv_beta1[l].reshape(1, -1), conv_W2[l],
            conv_b2[l].reshape(1, d), bn_g[l].reshape(1, d), bn_b[l].reshape(1, d),
            relu_out=(l < num_layers - 1),
        )
        if l < num_layers - 1:
            vn = _vn_call(gsum, vn, vW1[l], vb1[l], vg1[l], vbeta1[l],
                          vW2[l], vb2[l], vg2[l], vbeta2[l])
    return h


# speedup vs baseline: 3.7262x; 1.0551x over previous
import functools

import jax
import jax.numpy as jnp
from jax import lax
from jax.experimental import pallas as pl
from jax.experimental.pallas import tpu as pltpu
from jax.experimental.pallas import tpu_sc as plsc

F32 = jnp.float32
_BN_SCALE = 1.0 / (1.0 + 1e-5) ** 0.5


def _node_body(batch_ref, h_ref, vn_ref, hin_ref, gsum_ref):
    i = pl.program_id(0)
    b = batch_ref[...]
    iot = lax.broadcasted_iota(jnp.int32, (b.shape[0], vn_ref.shape[0]), 1)
    oh = (b == iot).astype(F32)
    hin = h_ref[...] + jnp.dot(oh, vn_ref[...], preferred_element_type=F32)
    hin_ref[...] = hin
    part = lax.dot_general(oh, hin, (((0,), (0,)), ((), ())),
                           preferred_element_type=F32)

    @pl.when(i == 0)
    def _():
        gsum_ref[...] = part

    @pl.when(i != 0)
    def _():
        gsum_ref[...] = gsum_ref[...] + part


def _node_call(batch2d, h, vn, bn=200):
    n, d = h.shape
    ng = vn.shape[0]
    grid = n // bn
    return pl.pallas_call(
        _node_body,
        grid=(grid,),
        in_specs=[
            pl.BlockSpec((bn, 1), lambda i: (i, 0)),
            pl.BlockSpec((bn, d), lambda i: (i, 0)),
            pl.BlockSpec((ng, d), lambda i: (0, 0)),
        ],
        out_specs=[
            pl.BlockSpec((bn, d), lambda i: (i, 0)),
            pl.BlockSpec((ng, d), lambda i: (0, 0)),
        ],
        out_shape=[
            jax.ShapeDtypeStruct((n, d), F32),
            jax.ShapeDtypeStruct((ng, d), F32),
        ],
    )(batch2d, h, vn)


def _edge_body(ea_ref, we_ref, be_ref, out_ref):
    out_ref[...] = (
        jnp.dot(ea_ref[...], we_ref[...], preferred_element_type=F32)
        + be_ref[...]
    )


def _edge_call(edge_attr, we, be2d, be_blk=2000):
    e, ed = edge_attr.shape
    d = we.shape[1]
    grid = e // be_blk
    return pl.pallas_call(
        _edge_body,
        grid=(grid,),
        in_specs=[
            pl.BlockSpec((be_blk, ed), lambda i: (i, 0)),
            pl.BlockSpec((ed, d), lambda i: (0, 0)),
            pl.BlockSpec((1, d), lambda i: (0, 0)),
        ],
        out_specs=pl.BlockSpec((be_blk, d), lambda i: (i, 0)),
        out_shape=jax.ShapeDtypeStruct((e, d), F32),
    )(edge_attr, we, be2d)


def _mlp_body(relu_out, hin_ref, a0_ref, a1_ref, eps_ref, w1_ref, b1_ref,
              g1_ref, beta1_ref, w2_ref, b2_ref, g_ref, bb_ref, out_ref):
    hin = hin_ref[...]
    pre = (1.0 + eps_ref[0, 0]) * hin + a0_ref[...] + a1_ref[...]
    t = jnp.dot(pre, w1_ref[...], preferred_element_type=F32) + b1_ref[...]
    t = jnp.maximum(t * _BN_SCALE * g1_ref[...] + beta1_ref[...], 0.0)
    h2 = jnp.dot(t, w2_ref[...], preferred_element_type=F32) + b2_ref[...]
    h2 = h2 * _BN_SCALE * g_ref[...] + bb_ref[...]
    if relu_out:
        h2 = jnp.maximum(h2, 0.0)
    out_ref[...] = h2 + hin


def _mlp_call(hin, aggout, eps11, w1, b1, g1, beta1, w2, b2, g, bb,
              relu_out, bn=200):
    n, d = hin.shape
    d2 = w1.shape[1]
    grid = n // bn
    nblk = n // bn
    return pl.pallas_call(
        functools.partial(_mlp_body, relu_out),
        grid=(grid,),
        in_specs=[
            pl.BlockSpec((bn, d), lambda i: (i, 0)),
            pl.BlockSpec((bn, d), lambda i: (i, 0)),
            pl.BlockSpec((bn, d), lambda i, nb=nblk: (i + nb, 0)),
            pl.BlockSpec((1, 1), lambda i: (0, 0)),
            pl.BlockSpec((d, d2), lambda i: (0, 0)),
            pl.BlockSpec((1, d2), lambda i: (0, 0)),
            pl.BlockSpec((1, d2), lambda i: (0, 0)),
            pl.BlockSpec((1, d2), lambda i: (0, 0)),
            pl.BlockSpec((d2, d), lambda i: (0, 0)),
            pl.BlockSpec((1, d), lambda i: (0, 0)),
            pl.BlockSpec((1, d), lambda i: (0, 0)),
            pl.BlockSpec((1, d), lambda i: (0, 0)),
        ],
        out_specs=pl.BlockSpec((bn, d), lambda i: (i, 0)),
        out_shape=jax.ShapeDtypeStruct((n, d), F32),
    )(hin, aggout, aggout, eps11, w1, b1, g1, beta1, w2, b2, g, bb)


def _post_body(batch_ref, hin_ref, a0_ref, a1_ref, eps_ref, w1_ref, b1_ref,
               g1_ref, beta1_ref, w2_ref, b2_ref, g_ref, bb_ref,
               gsum_ref, vnin_ref, vw1_ref, vb1_ref, vg1_ref, vbeta1_ref,
               vw2_ref, vb2_ref, vg2_ref, vbeta2_ref,
               hinout_ref, gsumout_ref, vnout_ref):
    i = pl.program_id(0)

    @pl.when(i == 0)
    def _():
        vt = gsum_ref[...] + vnin_ref[...]
        u = jnp.dot(vt, vw1_ref[...], preferred_element_type=F32) + vb1_ref[...]
        u = jnp.maximum(u * _BN_SCALE * vg1_ref[...] + vbeta1_ref[...], 0.0)
        u = jnp.dot(u, vw2_ref[...], preferred_element_type=F32) + vb2_ref[...]
        u = jnp.maximum(u * _BN_SCALE * vg2_ref[...] + vbeta2_ref[...], 0.0)
        vnout_ref[...] = vnin_ref[...] + u

    hin = hin_ref[...]
    pre = (1.0 + eps_ref[0, 0]) * hin + a0_ref[...] + a1_ref[...]
    t = jnp.dot(pre, w1_ref[...], preferred_element_type=F32) + b1_ref[...]
    t = jnp.maximum(t * _BN_SCALE * g1_ref[...] + beta1_ref[...], 0.0)
    h2 = jnp.dot(t, w2_ref[...], preferred_element_type=F32) + b2_ref[...]
    h2 = h2 * _BN_SCALE * g_ref[...] + bb_ref[...]
    h = jnp.maximum(h2, 0.0) + hin

    vnnew = vnout_ref[...]
    b = batch_ref[...]
    iot = lax.broadcasted_iota(jnp.int32, (b.shape[0], vnnew.shape[0]), 1)
    oh = (b == iot).astype(F32)
    hin_next = h + jnp.dot(oh, vnnew, preferred_element_type=F32)
    hinout_ref[...] = hin_next
    part = lax.dot_general(oh, hin_next, (((0,), (0,)), ((), ())),
                           preferred_element_type=F32)

    @pl.when(i == 0)
    def _():
        gsumout_ref[...] = part

    @pl.when(i != 0)
    def _():
        gsumout_ref[...] = gsumout_ref[...] + part


def _post_call(batch2d, hin, aggout, eps11, w1, b1, g1, beta1, w2, b2, g, bb,
               gsum, vnin, vw1, vb1, vg1, vbeta1, vw2, vb2, vg2, vbeta2,
               bn=200):
    n, d = hin.shape
    d2 = w1.shape[1]
    ng = vnin.shape[0]
    grid = n // bn
    nblk = n // bn
    full = lambda shape: pl.BlockSpec(shape, lambda i: (0, 0))
    return pl.pallas_call(
        _post_body,
        grid=(grid,),
        in_specs=[
            pl.BlockSpec((bn, 1), lambda i: (i, 0)),
            pl.BlockSpec((bn, d), lambda i: (i, 0)),
            pl.BlockSpec((bn, d), lambda i: (i, 0)),
            pl.BlockSpec((bn, d), lambda i, nb=nblk: (i + nb, 0)),
            full((1, 1)),
            full((d, d2)), full((1, d2)), full((1, d2)), full((1, d2)),
            full((d2, d)), full((1, d)), full((1, d)), full((1, d)),
            full((ng, d)), full((ng, d)),
            full((d, d2)), full((1, d2)), full((1, d2)), full((1, d2)),
            full((d2, d)), full((1, d)), full((1, d)), full((1, d)),
        ],
        out_specs=[
            pl.BlockSpec((bn, d), lambda i: (i, 0)),
            pl.BlockSpec((ng, d), lambda i: (0, 0)),
            pl.BlockSpec((ng, d), lambda i: (0, 0)),
        ],
        out_shape=[
            jax.ShapeDtypeStruct((n, d), F32),
            jax.ShapeDtypeStruct((ng, d), F32),
            jax.ShapeDtypeStruct((ng, d), F32),
        ],
    )(batch2d, hin, aggout, aggout, eps11, w1, b1, g1, beta1, w2, b2, g, bb,
      gsum, vnin, vw1, vb1, vg1, vbeta1, vw2, vb2, vg2, vbeta2)


def _vn_body(gsum_ref, vn_ref, w1_ref, b1_ref, g1_ref, beta1_ref, w2_ref,
             b2_ref, g2_ref, beta2_ref, out_ref):
    vt = gsum_ref[...] + vn_ref[...]
    u = jnp.dot(vt, w1_ref[...], preferred_element_type=F32) + b1_ref[...]
    u = jnp.maximum(u * _BN_SCALE * g1_ref[...] + beta1_ref[...], 0.0)
    u = jnp.dot(u, w2_ref[...], preferred_element_type=F32) + b2_ref[...]
    u = jnp.maximum(u * _BN_SCALE * g2_ref[...] + beta2_ref[...], 0.0)
    out_ref[...] = vn_ref[...] + u


def _vn_call(gsum, vn, w1, b1, g1, beta1, w2, b2, g2, beta2):
    ng, d = vn.shape
    d2 = w1.shape[1]
    return pl.pallas_call(
        _vn_body,
        out_shape=jax.ShapeDtypeStruct((ng, d), F32),
    )(gsum, vn, w1, b1.reshape(1, d2), g1.reshape(1, d2),
      beta1.reshape(1, d2), w2, b2.reshape(1, d), g2.reshape(1, d),
      beta2.reshape(1, d))


_CS = 64
_GRP = 32


def _make_sc_msg(n, d, e):
    cs = _CS
    grp = _GRP
    nchunk = e // cs
    nq = d // 16
    mesh = plsc.VectorSubcoreMesh(core_axis_name="c", subcore_axis_name="s")
    per_w = -(-nchunk // 32)
    per_w = ((per_w + grp - 1) // grp) * grp
    tail_w = nchunk - 31 * per_w
    assert 0 < tail_w <= per_w
    nfull = n // cs
    ztail = n - nfull * cs
    zfull_k = nfull // 16
    zrem = nfull % 16

    @functools.partial(
        pl.kernel,
        out_type=jax.ShapeDtypeStruct((2 * n, d), F32),
        mesh=mesh,
        scratch_types=[
            pltpu.VMEM((grp, cs), jnp.int32),
            pltpu.VMEM((grp, cs), jnp.int32),
            pltpu.VMEM((cs, d), F32),
            pltpu.VMEM((cs, d), F32),
            pltpu.VMEM((cs, d), F32),
            pltpu.VMEM((cs, d), F32),
            pltpu.VMEM((cs, d), F32),
            pltpu.VMEM_SHARED((n, d), F32),
            pltpu.SemaphoreType.DMA,
            pltpu.SemaphoreType.DMA,
            pltpu.SemaphoreType.DMA,
            pltpu.SemaphoreType.DMA,
            pltpu.SemaphoreType.DMA,
            pltpu.SemaphoreType.DMA,
            pltpu.SemaphoreType.DMA,
            pltpu.SemaphoreType.DMA,
        ],
    )
    def sc_msg(hin_hbm, emb_hbm, src_hbm, dst_hbm, out_hbm,
               srcblk, dstblk, r0, r1, r2, e0, e1, agg,
               gs0, gs1, gs2, es0, es1, ss0, ss1, ss2):
        c = lax.axis_index("c")
        s = lax.axis_index("s")
        w = c * 16 + s
        rbufs = (r0, r1, r2)
        ebufs = (e0, e1)
        gsems = (gs0, gs1, gs2)
        esems = (es0, es1)
        ssems = (ss0, ss1, ss2)

        zv = jnp.zeros((16,), F32)

        def zbody(r, carry):
            for q in range(nq):
                r0[r, pl.ds(q * 16, 16)] = zv
            return carry

        lax.fori_loop(0, cs, zbody, 0)

        nz = zfull_k + jnp.where(s < zrem, 1, 0)

        def zcopy(k, carry):
            z = s + 16 * k
            pltpu.sync_copy(r0.at[pl.ds(0, cs)], agg.at[pl.ds(z * cs, cs)])
            return carry

        lax.fori_loop(0, nz, zcopy, 0)

        @pl.when(s == 0)
        def _():
            pltpu.sync_copy(r0.at[pl.ds(0, ztail)],
                            agg.at[pl.ds(nfull * cs, ztail)])

        plsc.subcore_barrier()

        base = w * per_w
        nk = jnp.where(w == 31, tail_w, per_w)

        def issue_gather(m, j):
            return pltpu.async_copy(hin_hbm.at[srcblk.at[m]],
                                    rbufs[m % 3], gsems[m % 3])

        def issue_emb(m, j):
            return pltpu.async_copy(emb_hbm.at[pl.ds(j * cs, cs)],
                                    ebufs[m % 2], esems[m % 2])

        def gbody(g, carry):
            g0 = base + g * grp
            pltpu.sync_copy(src_hbm.at[pl.ds(g0, grp)], srcblk)
            pltpu.sync_copy(dst_hbm.at[pl.ds(g0, grp)], dstblk)
            nin = jnp.minimum(grp, nk - g * grp)

            issue_gather(0, g0)
            issue_emb(0, g0)

            @pl.when(1 < nin)
            def _():
                issue_gather(1, g0 + 1)
                issue_emb(1, g0 + 1)

            for kk in range(grp):
                rb = rbufs[kk % 3]
                eb = ebufs[kk % 2]

                @pl.when(kk < nin)
                def _(kk=kk, rb=rb, eb=eb):
                    pltpu.make_async_copy(
                        hin_hbm.at[srcblk.at[kk]], rb, gsems[kk % 3]).wait()
                    pltpu.make_async_copy(
                        emb_hbm.at[pl.ds((base + g * grp + kk) * cs, cs)],
                        eb, esems[kk % 2]).wait()
                    if kk >= 1:
                        pltpu.make_async_copy(
                            rbufs[(kk - 1) % 3],
                            agg.at[dstblk.at[kk - 1]],
                            ssems[(kk - 1) % 3]).wait()

                @pl.when(jnp.logical_and(kk < nin, kk + 2 < nin))
                def _(kk=kk):
                    issue_gather(kk + 2, g0 + kk + 2)

                @pl.when(kk < nin)
                def _(kk=kk, rb=rb, eb=eb):
                    def cbody(r2, cc):
                        for rr in range(2):
                            row = 2 * r2 + rr
                            for q in range(nq):
                                sl = pl.ds(q * 16, 16)
                                rb[row, sl] = jnp.maximum(
                                    rb[row, sl] + eb[row, sl], 0.0)
                        return cc

                    lax.fori_loop(0, cs // 2, cbody, 0)
                    if kk + 2 < grp:
                        @pl.when(kk + 2 < nin)
                        def _():
                            issue_emb(kk + 2, g0 + kk + 2)
                    pltpu.async_copy(rb, agg.at[dstblk.at[kk]],
                                     ssems[kk % 3], add=True)

                @pl.when(kk + 1 == nin)
                def _(kk=kk):
                    pltpu.make_async_copy(
                        rbufs[kk % 3], agg.at[dstblk.at[kk]],
                        ssems[kk % 3]).wait()

            return carry

        lax.fori_loop(0, (nk + grp - 1) // grp, gbody, 0)
        plsc.subcore_barrier()

        def wcopy(k, carry):
            z = s + 16 * k
            pltpu.sync_copy(agg.at[pl.ds(z * cs, cs)],
                            out_hbm.at[pl.ds(c * n + z * cs, cs)])
            return carry

        lax.fori_loop(0, nz, wcopy, 0)

        @pl.when(s == 0)
        def _():
            pltpu.sync_copy(agg.at[pl.ds(nfull * cs, ztail)],
                            out_hbm.at[pl.ds(c * n + nfull * cs, ztail)])

    return sc_msg


def kernel(input_feature, edge_index, edge_attr, batch, vn_emb, conv_We,
           conv_be, conv_eps, conv_W1, conv_b1, conv_g1, conv_beta1, conv_W2,
           conv_b2, bn_g, bn_b, vW1, vb1, vg1, vbeta1, vW2, vb2, vg2,
           vbeta2):
    n, d = input_feature.shape
    e = edge_index.shape[1]
    ng = 256
    num_layers = conv_We.shape[0]

    nchunk = e // _CS
    per_w = ((-(-nchunk // 32) + _GRP - 1) // _GRP) * _GRP
    pad_rows = 32 * per_w - nchunk
    src2d = jnp.pad(edge_index[0].reshape(nchunk, _CS), ((0, pad_rows), (0, 0)))
    dst2d = jnp.pad(edge_index[1].reshape(nchunk, _CS), ((0, pad_rows), (0, 0)))
    batch2d = batch.reshape(n, 1)
    vn = jnp.tile(vn_emb[0][None, :], (ng, 1))

    sc_msg = _make_sc_msg(n, d, e)

    embs = [_edge_call(edge_attr, conv_We[l], conv_be[l].reshape(1, d))
            for l in range(num_layers)]

    hin, gsum = _node_call(batch2d, input_feature, vn)
    for l in range(num_layers - 1):
        aggout = sc_msg(hin, embs[l], src2d, dst2d)
        hin, gsum, vn = _post_call(
            batch2d, hin, aggout, conv_eps[l].reshape(1, 1), conv_W1[l],
            conv_b1[l].reshape(1, -1), conv_g1[l].reshape(1, -1),
            conv_beta1[l].reshape(1, -1), conv_W2[l], conv_b2[l].reshape(1, d),
            bn_g[l].reshape(1, d), bn_b[l].reshape(1, d), gsum, vn,
            vW1[l], vb1[l].reshape(1, -1), vg1[l].reshape(1, -1),
            vbeta1[l].reshape(1, -1), vW2[l], vb2[l].reshape(1, d),
            vg2[l].reshape(1, d), vbeta2[l].reshape(1, d))
    l = num_layers - 1
    aggout = sc_msg(hin, embs[l], src2d, dst2d)
    return _mlp_call(
        hin, aggout, conv_eps[l].reshape(1, 1), conv_W1[l],
        conv_b1[l].reshape(1, -1), conv_g1[l].reshape(1, -1),
        conv_beta1[l].reshape(1, -1), conv_W2[l], conv_b2[l].reshape(1, d),
        bn_g[l].reshape(1, d), bn_b[l].reshape(1, d), relu_out=False)
